# baseline (speedup 1.0000x reference)
import jax
import jax.numpy as jnp
from jax.experimental import pallas as pl
from jax.experimental.pallas import tpu as pltpu

_NBINS = 1000
_NCLS = 91
_B = 8
_Q = 900
_N = _Q * _NCLS
_NB = 640
_BL = 128
_K = 100


def _postprocess_kernel(p3_ref, boxes_ref, ts_ref, conv_ref, scores_ref,
                        labels_ref, boxes_out_ref, pbuf, vcbuf, icbuf):
    shape3 = (_B, _NB, _BL)
    fi3 = (jax.lax.broadcasted_iota(jnp.int32, shape3, 1) * _BL
           + jax.lax.broadcasted_iota(jnp.int32, shape3, 2))
    prob = jnp.where(fi3 < _N, jax.nn.sigmoid(p3_ref[:, :, :]), -1.0)
    pbuf[:, :, :] = prob
    bm0 = jnp.max(prob, axis=2)

    col128 = jax.lax.broadcasted_iota(jnp.int32, (_B, 128), 1)
    biota = jax.lax.broadcasted_iota(jnp.int32, (_B, _NB), 1)
    big = jnp.int32(1 << 30)

    def sbody(i, carry):
        bm, jb = carry
        m = jnp.max(bm, axis=1, keepdims=True)
        j = jnp.min(jnp.where(bm == m, biota, big), axis=1,
                    keepdims=True)
        bm = jnp.where(biota == j, -1.0, bm)
        jb = jnp.where(col128 == i, j, jb)
        return bm, jb

    _, jb = jax.lax.fori_loop(
        0, _K, sbody, (bm0, jnp.zeros((_B, 128), jnp.int32)))

    bjiota = jax.lax.broadcasted_iota(jnp.int32, (_NB, _K), 0)
    idxpl = (jax.lax.broadcasted_iota(jnp.int32, (_NB, _BL), 0) * _BL
             + jax.lax.broadcasted_iota(jnp.int32, (_NB, _BL), 1)
             ).astype(jnp.float32)
    dnum = (((0,), (0,)), ((), ()))
    for b in range(_B):
        oh = (bjiota == jb[b:b + 1, :_K]).astype(jnp.float32)
        vcbuf[b] = jax.lax.dot_general(oh, pbuf[b], dnum,
                                       preferred_element_type=jnp.float32)
        icbuf[b] = jax.lax.dot_general(oh, idxpl, dnum,
                                       preferred_element_type=jnp.float32)

    bigf = jnp.float32(2.0e9)

    def mbody(i, carry):
        s_acc, l_acc, q_acc = carry
        vc = vcbuf[:, :, :]
        ic = icbuf[:, :, :]
        m = jnp.max(jnp.max(vc, axis=2, keepdims=True), axis=1,
                    keepdims=True)
        fidx = jnp.min(jnp.min(jnp.where(vc == m, ic, bigf), axis=2,
                               keepdims=True), axis=1, keepdims=True)
        vcbuf[:, :, :] = jnp.where(ic == fidx, -1.0, vc)
        colmask = col128 == i
        fi = fidx.reshape(_B, 1).astype(jnp.int32)
        s_acc = jnp.where(colmask, m.reshape(_B, 1), s_acc)
        l_acc = jnp.where(colmask, fi % _NCLS, l_acc)
        q_acc = jnp.where(colmask, fi // _NCLS, q_acc)
        return s_acc, l_acc, q_acc

    init = (jnp.zeros((_B, 128), jnp.float32),
            jnp.zeros((_B, 128), jnp.int32),
            jnp.zeros((_B, 128), jnp.int32))
    s_acc, l_acc, q_acc = jax.lax.fori_loop(0, _K, mbody, init)

    scores_ref[:, :] = s_acc[:, :_K]
    labels_ref[:, :] = l_acc[:, :_K]

    conv = conv_ref[:, :]
    qiota = jax.lax.broadcasted_iota(jnp.int32, (_Q, _K), 0)
    for b in range(_B):
        xyxy = jax.lax.dot(boxes_ref[b], conv,
                           preferred_element_type=jnp.float32)
        onehot_t = (qiota == q_acc[b:b + 1, :_K]).astype(jnp.float32)
        sel = jax.lax.dot_general(onehot_t, xyxy, dnum,
                                  preferred_element_type=jnp.float32)
        h = ts_ref[b:b + 1, 0:1]
        w = ts_ref[b:b + 1, 1:2]
        scale = jnp.concatenate([w, h, w, h], axis=1)
        boxes_out_ref[b] = sel * scale


def kernel(pred_logits, pred_boxes, target_sizes):
    flat = pred_logits[4, :, :, _NBINS:_NBINS + _NCLS].reshape(_B, _N)
    flat = jnp.pad(flat, ((0, 0), (0, _NB * _BL - _N)))
    p3 = flat.reshape(_B, _NB, _BL)
    conv = jnp.array([[1.0, 0.0, 1.0, 0.0],
                      [0.0, 1.0, 0.0, 1.0],
                      [-0.5, 0.0, 0.5, 0.0],
                      [0.0, -0.5, 0.0, 0.5]], dtype=jnp.float32)
    scores, labels, boxes = pl.pallas_call(
        _postprocess_kernel,
        out_shape=(
            jax.ShapeDtypeStruct((_B, _K), jnp.float32),
            jax.ShapeDtypeStruct((_B, _K), jnp.int32),
            jax.ShapeDtypeStruct((_B, _K, 4), jnp.float32),
        ),
        scratch_shapes=[
            pltpu.VMEM((_B, _NB, _BL), jnp.float32),
            pltpu.VMEM((_B, _K, _BL), jnp.float32),
            pltpu.VMEM((_B, _K, _BL), jnp.float32),
        ],
    )(p3, pred_boxes, target_sizes, conv)
    return scores, labels, boxes
